# baseline (speedup 1.0000x reference)
import jax
import jax.numpy as jnp
from jax import lax
from jax.experimental import pallas as pl
from jax.experimental.pallas import tpu as pltpu
from jax.experimental.pallas import tpu_sc as plsc

NUM_CORES = 2
NUM_SUBCORES = 16
LANES = 16
NW = NUM_CORES * NUM_SUBCORES

BATCH = 16384
EMBED_DIM = 32
B_PER_W = BATCH // NW
CHUNK = 128
N_CHUNK = B_PER_W // CHUNK
N_GRP = B_PER_W // LANES
W_PAD = 48


def _gmf_body(uid_hbm, iid_hbm, ut_hbm, it_hbm, w_hbm, b_hbm, out_hbm,
              idxu_v, idxi_v, gu_buf, gi_buf, w_v, b_v, out_v, sems):
    wid = lax.axis_index("c") * NUM_SUBCORES + lax.axis_index("s")
    crow = wid * N_CHUNK

    pltpu.sync_copy(uid_hbm.at[pl.ds(crow, N_CHUNK)], idxu_v)
    pltpu.sync_copy(iid_hbm.at[pl.ds(crow, N_CHUNK)], idxi_v)
    pltpu.sync_copy(w_hbm, w_v)
    pltpu.sync_copy(b_hbm, b_v)

    def copies(d, slot):
        cs = []
        for c in range(N_CHUNK):
            s = pl.ds(c * CHUNK, CHUNK)
            cs.append(pltpu.make_async_copy(
                ut_hbm.at[d].at[idxu_v.at[c]], gu_buf.at[slot, s],
                sems.at[slot]))
            cs.append(pltpu.make_async_copy(
                it_hbm.at[d].at[idxi_v.at[c]], gi_buf.at[slot, s],
                sems.at[slot]))
        return cs

    bias = b_v[...]
    for g in range(N_GRP):
        out_v[pl.ds(g * LANES, LANES)] = bias

    ones = jnp.full((LANES,), 1, jnp.int32)

    for c in copies(0, 0):
        c.start()

    def step(d, carry):
        slot = lax.rem(d, 2)

        @pl.when(d + 1 < EMBED_DIM)
        def _():
            for c in copies(d + 1, 1 - slot):
                c.start()

        for c in copies(d, slot):
            c.wait()

        w_d = plsc.load_gather(w_v, [ones * (d + 1)])
        for g in range(N_GRP):
            s = pl.ds(g * LANES, LANES)
            prod = (gu_buf[slot, s] * gi_buf[slot, s]) * w_d
            plsc.addupdate(out_v.at[s], prod)
        return carry

    lax.fori_loop(0, EMBED_DIM, step, 0)

    pltpu.sync_copy(out_v, out_hbm.at[pl.ds(wid * B_PER_W, B_PER_W)])


@jax.jit
def _gmf(user_ids, item_ids, ut_t, it_t, fc_w_pad, fc_b16):
    mesh = plsc.VectorSubcoreMesh(
        core_axis_name="c", subcore_axis_name="s",
        num_cores=NUM_CORES, num_subcores=NUM_SUBCORES)
    f = pl.kernel(
        _gmf_body,
        out_type=jax.ShapeDtypeStruct((BATCH,), jnp.float32),
        mesh=mesh,
        compiler_params=pltpu.CompilerParams(
            needs_layout_passes=False, use_tc_tiling_on_sc=False),
        scratch_types=[
            pltpu.VMEM((N_CHUNK, CHUNK), jnp.int32),
            pltpu.VMEM((N_CHUNK, CHUNK), jnp.int32),
            pltpu.VMEM((2, B_PER_W), jnp.float32),
            pltpu.VMEM((2, B_PER_W), jnp.float32),
            pltpu.VMEM((W_PAD,), jnp.float32),
            pltpu.VMEM((LANES,), jnp.float32),
            pltpu.VMEM((B_PER_W,), jnp.float32),
            pltpu.SemaphoreType.DMA((2,)),
        ],
    )
    return f(user_ids.reshape(BATCH // CHUNK, CHUNK),
             item_ids.reshape(BATCH // CHUNK, CHUNK),
             ut_t, it_t, fc_w_pad, fc_b16)


def kernel(user_ids, item_ids, user_table, item_table, fc_w, fc_b):
    w = fc_w.reshape(EMBED_DIM)
    fc_w_pad = jnp.zeros((W_PAD,), jnp.float32).at[1:EMBED_DIM + 1].set(w)
    fc_b16 = jnp.broadcast_to(fc_b, (LANES,))
    return _gmf(user_ids.astype(jnp.int32), item_ids.astype(jnp.int32),
                user_table.T, item_table.T, fc_w_pad, fc_b16)


# speedup vs baseline: 18.9700x; 18.9700x over previous
import jax
import jax.numpy as jnp
from jax import lax
from jax.experimental import pallas as pl
from jax.experimental.pallas import tpu as pltpu
from jax.experimental.pallas import tpu_sc as plsc

NUM_CORES = 2
NUM_SUBCORES = 16
LANES = 16
NW = NUM_CORES * NUM_SUBCORES

BATCH = 16384
EMBED_DIM = 32
B_PER_W = BATCH // NW
N_GRP = B_PER_W // LANES
SLOTS = 4
BLK = 128


def _gmf_body(uid_hbm, iid_hbm, ut_hbm, it_hbm, w_hbm, b_hbm, out_hbm,
              idxu_v, idxi_v, ubuf, ibuf, w_v, b_v, out_v, sems):
    wid = lax.axis_index("c") * NUM_SUBCORES + lax.axis_index("s")

    pltpu.sync_copy(uid_hbm.at[pl.ds(wid * N_GRP, N_GRP)], idxu_v)
    pltpu.sync_copy(iid_hbm.at[pl.ds(wid * N_GRP, N_GRP)], idxi_v)
    pltpu.sync_copy(w_hbm, w_v)
    pltpu.sync_copy(b_hbm, b_v)

    w_lo = w_v[pl.ds(0, LANES)]
    w_hi = w_v[pl.ds(LANES, LANES)]
    bias = b_v[...]
    lane_iota = lax.broadcasted_iota(jnp.int32, (LANES,), 0)
    d_lo = lane_iota
    d_hi = lane_iota + LANES
    ones = jnp.full((LANES,), 1, jnp.int32)

    def group(g, carry):
        vu = idxu_v[g]
        vi = idxi_v[g]

        fired = {}

        def fire(l):
            slot = l % SLOTS
            r_u = vu[l]
            r_i = vi[l]
            cu = pl.multiple_of(
                lax.shift_left(lax.shift_right_logical(r_u, 7), 7), BLK)
            ci = pl.multiple_of(
                lax.shift_left(lax.shift_right_logical(r_i, 7), 7), BLK)
            cp_u = pltpu.make_async_copy(
                ut_hbm.at[:, pl.ds(cu, BLK)], ubuf.at[slot], sems.at[slot])
            cp_i = pltpu.make_async_copy(
                it_hbm.at[:, pl.ds(ci, BLK)], ibuf.at[slot], sems.at[slot])
            cp_u.start()
            cp_i.start()
            fired[l] = (cp_u, cp_i, r_u & 127, r_i & 127)

        def consume(l, res):
            slot = l % SLOTS
            cp_u, cp_i, lu, li = fired.pop(l)
            cp_u.wait()
            cp_i.wait()
            lane_u = ones * lu
            lane_i = ones * li
            ug_lo = plsc.load_gather(ubuf.at[slot], [d_lo, lane_u])
            ug_hi = plsc.load_gather(ubuf.at[slot], [d_hi, lane_u])
            ig_lo = plsc.load_gather(ibuf.at[slot], [d_lo, lane_i])
            ig_hi = plsc.load_gather(ibuf.at[slot], [d_hi, lane_i])
            t = (ug_lo * ig_lo) * w_lo + (ug_hi * ig_hi) * w_hi
            s = jnp.sum(t)
            return jnp.where(lane_iota == l, s, res)

        res = jnp.zeros((LANES,), jnp.float32)
        for l in range(LANES):
            fire(l)
            if l >= SLOTS - 1:
                res = consume(l - (SLOTS - 1), res)
        for l in range(LANES - (SLOTS - 1), LANES):
            res = consume(l, res)

        out_v[pl.ds(g * LANES, LANES)] = res + bias
        return carry

    lax.fori_loop(0, N_GRP, group, 0)

    pltpu.sync_copy(out_v, out_hbm.at[pl.ds(wid * B_PER_W, B_PER_W)])


@jax.jit
def _gmf(user_ids, item_ids, ut_t, it_t, fc_w32, fc_b16):
    mesh = plsc.VectorSubcoreMesh(
        core_axis_name="c", subcore_axis_name="s",
        num_cores=NUM_CORES, num_subcores=NUM_SUBCORES)
    f = pl.kernel(
        _gmf_body,
        out_type=jax.ShapeDtypeStruct((BATCH,), jnp.float32),
        mesh=mesh,
        compiler_params=pltpu.CompilerParams(needs_layout_passes=False),
        scratch_types=[
            pltpu.VMEM((N_GRP, LANES), jnp.int32),
            pltpu.VMEM((N_GRP, LANES), jnp.int32),
            pltpu.VMEM((SLOTS, EMBED_DIM, BLK), jnp.float32),
            pltpu.VMEM((SLOTS, EMBED_DIM, BLK), jnp.float32),
            pltpu.VMEM((EMBED_DIM,), jnp.float32),
            pltpu.VMEM((LANES,), jnp.float32),
            pltpu.VMEM((B_PER_W,), jnp.float32),
            pltpu.SemaphoreType.DMA((SLOTS,)),
        ],
    )
    return f(user_ids.reshape(BATCH // LANES, LANES),
             item_ids.reshape(BATCH // LANES, LANES),
             ut_t, it_t, fc_w32, fc_b16)


def kernel(user_ids, item_ids, user_table, item_table, fc_w, fc_b):
    fc_w32 = fc_w.reshape(EMBED_DIM)
    fc_b16 = jnp.broadcast_to(fc_b, (LANES,))
    return _gmf(user_ids.astype(jnp.int32), item_ids.astype(jnp.int32),
                user_table.T, item_table.T, fc_w32, fc_b16)


# speedup vs baseline: 21.2846x; 1.1220x over previous
import jax
import jax.numpy as jnp
from jax import lax
from jax.experimental import pallas as pl
from jax.experimental.pallas import tpu as pltpu
from jax.experimental.pallas import tpu_sc as plsc

NUM_CORES = 2
NUM_SUBCORES = 16
LANES = 16
NW = NUM_CORES * NUM_SUBCORES

BATCH = 16384
EMBED_DIM = 32
B_PER_W = BATCH // NW
N_GRP = B_PER_W // LANES
SLOTS = 8
BLK = 128


def _gmf_body(uid_hbm, iid_hbm, ut_hbm, it_hbm, w_hbm, b_hbm, out_hbm,
              idxu_v, idxi_v, ubuf, ibuf, w_v, b_v, out_v, sems):
    wid = lax.axis_index("c") * NUM_SUBCORES + lax.axis_index("s")

    pltpu.sync_copy(uid_hbm.at[pl.ds(wid * N_GRP, N_GRP)], idxu_v)
    pltpu.sync_copy(iid_hbm.at[pl.ds(wid * N_GRP, N_GRP)], idxi_v)
    pltpu.sync_copy(w_hbm, w_v)
    pltpu.sync_copy(b_hbm, b_v)

    w_lo = w_v[pl.ds(0, LANES)]
    w_hi = w_v[pl.ds(LANES, LANES)]
    bias = b_v[...]
    lane_iota = lax.broadcasted_iota(jnp.int32, (LANES,), 0)
    d_lo = lane_iota
    d_hi = lane_iota + LANES
    ones = jnp.full((LANES,), 1, jnp.int32)

    def group(g, carry):
        vu = idxu_v[g]
        vi = idxi_v[g]

        fired = {}

        def fire(l):
            slot = l % SLOTS
            r_u = vu[l]
            r_i = vi[l]
            cu = pl.multiple_of(
                lax.shift_left(lax.shift_right_logical(r_u, 7), 7), BLK)
            ci = pl.multiple_of(
                lax.shift_left(lax.shift_right_logical(r_i, 7), 7), BLK)
            cp_u = pltpu.make_async_copy(
                ut_hbm.at[:, pl.ds(cu, BLK)], ubuf.at[slot], sems.at[slot])
            cp_i = pltpu.make_async_copy(
                it_hbm.at[:, pl.ds(ci, BLK)], ibuf.at[slot], sems.at[slot])
            cp_u.start()
            cp_i.start()
            fired[l] = (cp_u, cp_i, r_u & 127, r_i & 127)

        def consume(l, res):
            slot = l % SLOTS
            cp_u, cp_i, lu, li = fired.pop(l)
            cp_u.wait()
            cp_i.wait()
            lane_u = ones * lu
            lane_i = ones * li
            ug_lo = plsc.load_gather(ubuf.at[slot], [d_lo, lane_u])
            ug_hi = plsc.load_gather(ubuf.at[slot], [d_hi, lane_u])
            ig_lo = plsc.load_gather(ibuf.at[slot], [d_lo, lane_i])
            ig_hi = plsc.load_gather(ibuf.at[slot], [d_hi, lane_i])
            t = (ug_lo * ig_lo) * w_lo + (ug_hi * ig_hi) * w_hi
            s = jnp.sum(t)
            return jnp.where(lane_iota == l, s, res)

        res = jnp.zeros((LANES,), jnp.float32)
        for l in range(LANES):
            fire(l)
            if l >= SLOTS - 1:
                res = consume(l - (SLOTS - 1), res)
        for l in range(LANES - (SLOTS - 1), LANES):
            res = consume(l, res)

        out_v[pl.ds(g * LANES, LANES)] = res + bias
        return carry

    lax.fori_loop(0, N_GRP, group, 0)

    pltpu.sync_copy(out_v, out_hbm.at[pl.ds(wid * B_PER_W, B_PER_W)])


@jax.jit
def _gmf(user_ids, item_ids, ut_t, it_t, fc_w32, fc_b16):
    mesh = plsc.VectorSubcoreMesh(
        core_axis_name="c", subcore_axis_name="s",
        num_cores=NUM_CORES, num_subcores=NUM_SUBCORES)
    f = pl.kernel(
        _gmf_body,
        out_type=jax.ShapeDtypeStruct((BATCH,), jnp.float32),
        mesh=mesh,
        compiler_params=pltpu.CompilerParams(needs_layout_passes=False),
        scratch_types=[
            pltpu.VMEM((N_GRP, LANES), jnp.int32),
            pltpu.VMEM((N_GRP, LANES), jnp.int32),
            pltpu.VMEM((SLOTS, EMBED_DIM, BLK), jnp.float32),
            pltpu.VMEM((SLOTS, EMBED_DIM, BLK), jnp.float32),
            pltpu.VMEM((EMBED_DIM,), jnp.float32),
            pltpu.VMEM((LANES,), jnp.float32),
            pltpu.VMEM((B_PER_W,), jnp.float32),
            pltpu.SemaphoreType.DMA((SLOTS,)),
        ],
    )
    return f(user_ids.reshape(BATCH // LANES, LANES),
             item_ids.reshape(BATCH // LANES, LANES),
             ut_t, it_t, fc_w32, fc_b16)


def kernel(user_ids, item_ids, user_table, item_table, fc_w, fc_b):
    fc_w32 = fc_w.reshape(EMBED_DIM)
    fc_b16 = jnp.broadcast_to(fc_b, (LANES,))
    return _gmf(user_ids.astype(jnp.int32), item_ids.astype(jnp.int32),
                user_table.T, item_table.T, fc_w32, fc_b16)


# speedup vs baseline: 24.1440x; 1.1343x over previous
import jax
import jax.numpy as jnp
from jax import lax
from jax.experimental import pallas as pl
from jax.experimental.pallas import tpu as pltpu
from jax.experimental.pallas import tpu_sc as plsc

NUM_CORES = 2
NUM_SUBCORES = 16
LANES = 16
NW = NUM_CORES * NUM_SUBCORES

BATCH = 16384
EMBED_DIM = 32
B_PER_W = BATCH // NW
N_GRP = B_PER_W // LANES
SLOTS = 8
N_STEP = B_PER_W // SLOTS
BLK = 128


def _gmf_body(uid_hbm, iid_hbm, ut_hbm, it_hbm, w_hbm, b_hbm, out_hbm,
              idxu_v, idxi_v, ubuf, ibuf, w_v, b_v, out_v, sems):
    wid = lax.axis_index("c") * NUM_SUBCORES + lax.axis_index("s")

    pltpu.sync_copy(uid_hbm.at[pl.ds(wid * N_GRP, N_GRP)], idxu_v)
    pltpu.sync_copy(iid_hbm.at[pl.ds(wid * N_GRP, N_GRP)], idxi_v)
    pltpu.sync_copy(w_hbm, w_v)
    pltpu.sync_copy(b_hbm, b_v)

    w_lo = w_v[pl.ds(0, LANES)]
    w_hi = w_v[pl.ds(LANES, LANES)]
    bias = b_v[...]
    lane_iota = lax.broadcasted_iota(jnp.int32, (LANES,), 0)
    d_lo = lane_iota
    d_hi = lane_iota + LANES
    ones = jnp.full((LANES,), 1, jnp.int32)
    zf = jnp.zeros((LANES,), jnp.float32)
    zi = jnp.zeros((LANES,), jnp.int32)

    def seed(g, carry):
        out_v[pl.ds(g * LANES, LANES)] = bias
        return carry
    lax.fori_loop(0, N_GRP, seed, 0)

    def fire(k, t):
        grp = lax.shift_right_logical(t, 1)
        pos = (t & 1) * SLOTS + k
        pos_splat = ones * pos
        vu = idxu_v[grp]
        vi = idxi_v[grp]
        r_u = jnp.sum(jnp.where(lane_iota == pos_splat, vu, zi))
        r_i = jnp.sum(jnp.where(lane_iota == pos_splat, vi, zi))
        cu = pl.multiple_of(
            lax.shift_left(lax.shift_right_logical(r_u, 7), 7), BLK)
        ci = pl.multiple_of(
            lax.shift_left(lax.shift_right_logical(r_i, 7), 7), BLK)
        pltpu.make_async_copy(
            ut_hbm.at[:, pl.ds(cu, BLK)], ubuf.at[k], sems.at[k]).start()
        pltpu.make_async_copy(
            it_hbm.at[:, pl.ds(ci, BLK)], ibuf.at[k], sems.at[k]).start()
        return r_u & 127, r_i & 127

    def consume(k, t, lu, li):
        pltpu.make_async_copy(
            ut_hbm.at[:, pl.ds(0, BLK)], ubuf.at[k], sems.at[k]).wait()
        pltpu.make_async_copy(
            it_hbm.at[:, pl.ds(0, BLK)], ibuf.at[k], sems.at[k]).wait()
        lane_u = ones * lu
        lane_i = ones * li
        ug_lo = plsc.load_gather(ubuf.at[k], [d_lo, lane_u])
        ug_hi = plsc.load_gather(ubuf.at[k], [d_hi, lane_u])
        ig_lo = plsc.load_gather(ibuf.at[k], [d_lo, lane_i])
        ig_hi = plsc.load_gather(ibuf.at[k], [d_hi, lane_i])
        t_v = (ug_lo * ig_lo) * w_lo + (ug_hi * ig_hi) * w_hi
        s = jnp.sum(t_v)
        grp = lax.shift_right_logical(t, 1)
        pos = (t & 1) * SLOTS + k
        contrib = jnp.where(lane_iota == ones * pos, s, zf)
        plsc.addupdate(out_v.at[pl.ds(grp * LANES, LANES)], contrib)

    carry0 = []
    for k in range(SLOTS):
        lu, li = fire(k, jnp.int32(0))
        carry0.extend((lu, li))

    def step(t, carry):
        new = list(carry)
        for k in range(SLOTS):
            consume(k, t - 1, carry[2 * k], carry[2 * k + 1])
            lu, li = fire(k, t)
            new[2 * k] = lu
            new[2 * k + 1] = li
        return tuple(new)

    carry = lax.fori_loop(1, N_STEP, step, tuple(carry0))

    for k in range(SLOTS):
        consume(k, jnp.int32(N_STEP - 1), carry[2 * k], carry[2 * k + 1])

    pltpu.sync_copy(out_v, out_hbm.at[pl.ds(wid * B_PER_W, B_PER_W)])


@jax.jit
def _gmf(user_ids, item_ids, ut_t, it_t, fc_w32, fc_b16):
    mesh = plsc.VectorSubcoreMesh(
        core_axis_name="c", subcore_axis_name="s",
        num_cores=NUM_CORES, num_subcores=NUM_SUBCORES)
    f = pl.kernel(
        _gmf_body,
        out_type=jax.ShapeDtypeStruct((BATCH,), jnp.float32),
        mesh=mesh,
        compiler_params=pltpu.CompilerParams(needs_layout_passes=False),
        scratch_types=[
            pltpu.VMEM((N_GRP, LANES), jnp.int32),
            pltpu.VMEM((N_GRP, LANES), jnp.int32),
            pltpu.VMEM((SLOTS, EMBED_DIM, BLK), jnp.float32),
            pltpu.VMEM((SLOTS, EMBED_DIM, BLK), jnp.float32),
            pltpu.VMEM((EMBED_DIM,), jnp.float32),
            pltpu.VMEM((LANES,), jnp.float32),
            pltpu.VMEM((B_PER_W,), jnp.float32),
            pltpu.SemaphoreType.DMA((SLOTS,)),
        ],
    )
    return f(user_ids.reshape(BATCH // LANES, LANES),
             item_ids.reshape(BATCH // LANES, LANES),
             ut_t, it_t, fc_w32, fc_b16)


def kernel(user_ids, item_ids, user_table, item_table, fc_w, fc_b):
    fc_w32 = fc_w.reshape(EMBED_DIM)
    fc_b16 = jnp.broadcast_to(fc_b, (LANES,))
    return _gmf(user_ids.astype(jnp.int32), item_ids.astype(jnp.int32),
                user_table.T, item_table.T, fc_w32, fc_b16)
